# baseline (speedup 1.0000x reference)
import functools

import jax
import jax.numpy as jnp
from jax import lax
from jax.experimental import pallas as pl
from jax.experimental.pallas import tpu as pltpu
from jax.experimental.pallas import tpu_sc as plsc

_E = 8
_TM = 512
_LOSS_SCALE = 3e-06
_PHI = lax.Precision.HIGHEST


def _router_body(lg_ref, p_ref, mp_ref, be_ref, nblk_ref, loss_ref):
    T = lg_ref.shape[1]
    lg = lg_ref[...]
    erow = lax.broadcasted_iota(jnp.int32, (_E, T), 0)
    m = jnp.max(lg, axis=0, keepdims=True)
    ex = jnp.exp(lg - m)
    ssum = jnp.sum(ex, axis=0, keepdims=True)
    pr = ex / ssum
    mpc = jnp.max(pr, axis=0, keepdims=True)
    idx = jnp.min(jnp.where(pr == mpc, erow, jnp.int32(999)),
                  axis=0, keepdims=True)
    oh = (erow == idx).astype(jnp.float32)
    mp = mpc

    C = 128
    G = T // C
    iu = lax.broadcasted_iota(jnp.int32, (C, C), 0)
    ju = lax.broadcasted_iota(jnp.int32, (C, C), 1)
    U = (iu <= ju).astype(jnp.float32)
    chunks = [
        lax.dot_general(oh[:, g * C:(g + 1) * C], U, (((1,), (0,)), ((), ())),
                        preferred_element_type=jnp.float32, precision=_PHI)
        for g in range(G)
    ]
    intra = jnp.concatenate(chunks, axis=1)
    tg = lax.broadcasted_iota(jnp.int32, (T, G), 0) // C
    gg = lax.broadcasted_iota(jnp.int32, (T, G), 1)
    S = (tg == gg).astype(jnp.float32)
    totals = lax.dot_general(oh, S, (((1,), (0,)), ((), ())),
                             preferred_element_type=jnp.float32, precision=_PHI)
    ig = lax.broadcasted_iota(jnp.int32, (G, G), 0)
    jg = lax.broadcasted_iota(jnp.int32, (G, G), 1)
    UgS = (ig < jg).astype(jnp.float32)
    offs = lax.dot_general(totals, UgS, (((1,), (0,)), ((), ())),
                           preferred_element_type=jnp.float32, precision=_PHI)
    St_g = lax.broadcasted_iota(jnp.int32, (G, T), 0)
    St_t = lax.broadcasted_iota(jnp.int32, (G, T), 1) // C
    St = (St_g == St_t).astype(jnp.float32)
    offs_full = lax.dot_general(offs, St, (((1,), (0,)), ((), ())),
                                preferred_element_type=jnp.float32, precision=_PHI)
    cum = intra + offs_full

    rank_incl = jnp.sum(cum * oh, axis=0, keepdims=True)
    counts = cum[:, T - 1:T]
    nb = jnp.floor((counts + (_TM - 1)) / _TM)
    il = lax.broadcasted_iota(jnp.int32, (_E, _E), 0)
    jl = lax.broadcasted_iota(jnp.int32, (_E, _E), 1)
    L8s = (jl < il).astype(jnp.float32)
    po = lax.dot_general(L8s, nb, (((1,), (0,)), ((), ())),
                         preferred_element_type=jnp.float32, precision=_PHI) * _TM
    psel = jnp.sum(po * oh, axis=0, keepdims=True)
    p = psel + rank_incl - 1.0
    p_ref[...] = p.astype(jnp.int32)
    mp_ref[...] = mp

    maxb = be_ref.shape[1]
    nbi = po / _TM + nb
    jbf = lax.broadcasted_iota(jnp.int32, (_E, maxb), 1).astype(jnp.float32)
    becnt = jnp.sum((nbi <= jbf).astype(jnp.float32), axis=0, keepdims=True)
    be_ref[...] = jnp.minimum(becnt, float(_E - 1)).astype(jnp.int32)
    nblk_ref[...] = jnp.full((1, 1), 1.0, jnp.float32).astype(jnp.int32) * \
        jnp.sum(nb).astype(jnp.int32)

    pe = lax.dot_general(oh, mp, (((1,), (1,)), ((), ())),
                         preferred_element_type=jnp.float32, precision=_PHI)
    loss = jnp.sum(counts * pe) * (_LOSS_SCALE * _E / (T * float(T) * T))
    loss_ref[...] = jnp.full((1, 1), 1.0, jnp.float32) * loss


def _router_call(lgp, maxb):
    T = lgp.shape[1]
    return pl.pallas_call(
        _router_body,
        out_shape=(
            jax.ShapeDtypeStruct((1, T), jnp.int32),
            jax.ShapeDtypeStruct((1, T), jnp.float32),
            jax.ShapeDtypeStruct((1, maxb), jnp.int32),
            jax.ShapeDtypeStruct((1, 1), jnp.int32),
            jax.ShapeDtypeStruct((1, 1), jnp.float32),
        ),
    )(lgp)


def _sc_wid():
    return lax.axis_index("s") * 2 + lax.axis_index("c")


def _dispatch_call(xt, p, mpb, padT):
    T, D = xt.shape
    NW = 32
    rpw = T // NW
    mesh = plsc.VectorSubcoreMesh(core_axis_name="c", subcore_axis_name="s")

    CH = 32
    nch = rpw // CH

    @functools.partial(
        pl.kernel, mesh=mesh,
        out_type=[jax.ShapeDtypeStruct((padT, D), jnp.int32),
                  jax.ShapeDtypeStruct((padT, 128), jnp.float32)],
        scratch_types=[pltpu.VMEM((rpw,), jnp.int32),
                       pltpu.VMEM((CH, D), jnp.int32),
                       pltpu.VMEM((CH, D), jnp.int32),
                       pltpu.VMEM((CH, 128), jnp.float32),
                       pltpu.VMEM((CH, 128), jnp.float32),
                       pltpu.SemaphoreType.DMA,
                       pltpu.SemaphoreType.DMA,
                       pltpu.SemaphoreType.DMA,
                       pltpu.SemaphoreType.DMA,
                       pltpu.SemaphoreType.DMA,
                       pltpu.SemaphoreType.DMA,
                       pltpu.SemaphoreType.DMA,
                       pltpu.SemaphoreType.DMA,
                       pltpu.SemaphoreType.DMA],
    )
    def k(x_hbm, p_hbm, mp_hbm, xs_hbm, mps_hbm, idx_v, rows0, rows1,
          mp0, mp1, li, lr0, lr1, lm0, lm1, sx0, sx1, sm0, sm1):
        b = _sc_wid() * rpw
        pltpu.async_copy(p_hbm.at[pl.ds(b, rpw)], idx_v, li).wait()
        rows = (rows0, rows1)
        mpv = (mp0, mp1)
        lrs = (lr0, lr1)
        lms = (lm0, lm1)
        sxs = (sx0, sx1)
        sms = (sm0, sm1)

        def loads(c):
            sl = c % 2
            return (
                pltpu.async_copy(x_hbm.at[pl.ds(b + c * CH, CH)],
                                 rows[sl], lrs[sl]),
                pltpu.async_copy(mp_hbm.at[pl.ds(b + c * CH, CH)],
                                 mpv[sl], lms[sl]))

        ld = [None] * nch
        sc = [None] * nch
        ld[0] = loads(0)
        ld[1] = loads(1)
        for c in range(nch):
            sl = c % 2
            ld[c][0].wait()
            ld[c][1].wait()
            isl = idx_v.at[pl.ds(c * CH, CH)]
            sc[c] = (pltpu.async_copy(rows[sl], xs_hbm.at[isl], sxs[sl]),
                     pltpu.async_copy(mpv[sl], mps_hbm.at[isl], sms[sl]))
            if c + 2 < nch:
                sc[c][0].wait()
                sc[c][1].wait()
                ld[c + 2] = loads(c + 2)
        for c in range(max(0, nch - 2), nch):
            sc[c][0].wait()
            sc[c][1].wait()

    return k(xt, p, mpb)


def _gmm_body(be_ref, nb_ref, xs_ref, w_ref, b_ref, mps_ref, ys_ref):
    @pl.when(pl.program_id(0) < nb_ref[0, 0])
    def _():
        w32 = xs_ref[...]
        Dh = w32.shape[1]
        lo = lax.bitcast_convert_type(w32 << 16, jnp.float32)
        hi = lax.bitcast_convert_type(w32 & jnp.int32(-65536), jnp.float32)
        acc = lax.dot_general(lo, w_ref[0][:, :Dh], (((1,), (1,)), ((), ())),
                              preferred_element_type=jnp.float32)
        acc = acc + lax.dot_general(hi, w_ref[0][:, Dh:],
                                    (((1,), (1,)), ((), ())),
                                    preferred_element_type=jnp.float32)
        ys_ref[...] = (acc + b_ref[0]) * mps_ref[:, :1]


def _gmm_call(be, nblk, xs, expert_W, expert_b, mps, maxb):
    padT, Dh = xs.shape
    D = expert_W.shape[1]
    grid_spec = pltpu.PrefetchScalarGridSpec(
        num_scalar_prefetch=2,
        grid=(maxb,),
        in_specs=[
            pl.BlockSpec((_TM, Dh),
                         lambda j, be, nb: (jnp.minimum(j, nb[0, 0] - 1), 0)),
            pl.BlockSpec((1, D, D), lambda j, be, nb: (be[0, j], 0, 0)),
            pl.BlockSpec((1, 1, D), lambda j, be, nb: (be[0, j], 0, 0)),
            pl.BlockSpec((_TM, 128),
                         lambda j, be, nb: (jnp.minimum(j, nb[0, 0] - 1), 0)),
        ],
        out_specs=pl.BlockSpec(
            (_TM, D), lambda j, be, nb: (jnp.minimum(j, nb[0, 0] - 1), 0)),
    )
    return pl.pallas_call(
        _gmm_body,
        grid_spec=grid_spec,
        out_shape=jax.ShapeDtypeStruct((padT, D), jnp.float32),
    )(be, nblk, xs, expert_W, expert_b, mps)


def _combine_call(ys, p):
    padT, D = ys.shape
    T = p.shape[0]
    NW = 32
    rpw = T // NW
    CH = 32
    mesh = plsc.VectorSubcoreMesh(core_axis_name="c", subcore_axis_name="s")

    @functools.partial(
        pl.kernel, mesh=mesh,
        out_type=jax.ShapeDtypeStruct((T, D), jnp.float32),
        scratch_types=[pltpu.VMEM((rpw,), jnp.int32),
                       pltpu.VMEM((CH, D), jnp.float32),
                       pltpu.VMEM((CH, D), jnp.float32),
                       pltpu.SemaphoreType.DMA,
                       pltpu.SemaphoreType.DMA,
                       pltpu.SemaphoreType.DMA,
                       pltpu.SemaphoreType.DMA,
                       pltpu.SemaphoreType.DMA],
    )
    def k(ys_hbm, p_hbm, out_hbm, idx_v, rows0, rows1, li, g0, g1, st0, st1):
        b = _sc_wid() * rpw
        pltpu.async_copy(p_hbm.at[pl.ds(b, rpw)], idx_v, li).wait()
        rows = (rows0, rows1)
        gsem = (g0, g1)
        ssem = (st0, st1)
        nch = rpw // CH
        gh = [None] * nch
        sh = [None] * nch
        gh[0] = pltpu.async_copy(ys_hbm.at[idx_v.at[pl.ds(0, CH)]], rows0, g0)
        for c in range(nch):
            sl = c % 2
            if c >= 1:
                sh[c - 1].wait()
            if c + 1 < nch:
                sl1 = (c + 1) % 2
                gh[c + 1] = pltpu.async_copy(
                    ys_hbm.at[idx_v.at[pl.ds((c + 1) * CH, CH)]],
                    rows[sl1], gsem[sl1])
            gh[c].wait()
            sh[c] = pltpu.async_copy(rows[sl], out_hbm.at[pl.ds(b + c * CH, CH)],
                                     ssem[sl])
        sh[nch - 1].wait()

    return k(ys, p)


def kernel(x, router_W, router_b, expert_W, expert_b):
    B, S, D = x.shape
    T = B * S
    maxb = T // _TM + _E - 1
    padT = maxb * _TM
    xt = x.reshape(T, D)
    logits = xt @ router_W.T + router_b

    p_row, mp_row, be_row, nblk11, loss11 = _router_call(logits.T, maxb)
    p = p_row.reshape(T)
    mpb = jnp.broadcast_to(mp_row.reshape(T, 1), (T, 128))

    xb = xt.astype(jnp.bfloat16)
    xpk = lax.bitcast_convert_type(
        jnp.stack([xb[:, :D // 2], xb[:, D // 2:]], axis=-1), jnp.int32)
    xs, mps = _dispatch_call(xpk, p, mpb, padT)
    ys = _gmm_call(be_row, nblk11, xs, expert_W, expert_b.reshape(_E, 1, D),
                   mps, maxb)
    out = _combine_call(ys, p)
    return out.reshape(B, S, D), loss11.reshape(())


# speedup vs baseline: 1.1343x; 1.1343x over previous
import functools

import jax
import jax.numpy as jnp
from jax import lax
from jax.experimental import pallas as pl
from jax.experimental.pallas import tpu as pltpu
from jax.experimental.pallas import tpu_sc as plsc

_E = 8
_TM = 512
_LOSS_SCALE = 3e-06
_PHI = lax.Precision.HIGHEST


def _router_body(lg_ref, p_ref, mp_ref, be_ref, nblk_ref, loss_ref):
    T = lg_ref.shape[1]
    lg = lg_ref[...]
    erow = lax.broadcasted_iota(jnp.int32, (_E, T), 0)
    m = jnp.max(lg, axis=0, keepdims=True)
    ex = jnp.exp(lg - m)
    ssum = jnp.sum(ex, axis=0, keepdims=True)
    pr = ex / ssum
    mpc = jnp.max(pr, axis=0, keepdims=True)
    idx = jnp.min(jnp.where(pr == mpc, erow, jnp.int32(999)),
                  axis=0, keepdims=True)
    oh = (erow == idx).astype(jnp.float32)
    mp = mpc

    C = 128
    G = T // C
    iu = lax.broadcasted_iota(jnp.int32, (C, C), 0)
    ju = lax.broadcasted_iota(jnp.int32, (C, C), 1)
    U = (iu <= ju).astype(jnp.float32)
    chunks = [
        lax.dot_general(oh[:, g * C:(g + 1) * C], U, (((1,), (0,)), ((), ())),
                        preferred_element_type=jnp.float32, precision=_PHI)
        for g in range(G)
    ]
    intra = jnp.concatenate(chunks, axis=1)
    tg = lax.broadcasted_iota(jnp.int32, (T, G), 0) // C
    gg = lax.broadcasted_iota(jnp.int32, (T, G), 1)
    S = (tg == gg).astype(jnp.float32)
    totals = lax.dot_general(oh, S, (((1,), (0,)), ((), ())),
                             preferred_element_type=jnp.float32, precision=_PHI)
    ig = lax.broadcasted_iota(jnp.int32, (G, G), 0)
    jg = lax.broadcasted_iota(jnp.int32, (G, G), 1)
    UgS = (ig < jg).astype(jnp.float32)
    offs = lax.dot_general(totals, UgS, (((1,), (0,)), ((), ())),
                           preferred_element_type=jnp.float32, precision=_PHI)
    St_g = lax.broadcasted_iota(jnp.int32, (G, T), 0)
    St_t = lax.broadcasted_iota(jnp.int32, (G, T), 1) // C
    St = (St_g == St_t).astype(jnp.float32)
    offs_full = lax.dot_general(offs, St, (((1,), (0,)), ((), ())),
                                preferred_element_type=jnp.float32, precision=_PHI)
    cum = intra + offs_full

    rank_incl = jnp.sum(cum * oh, axis=0, keepdims=True)
    counts = cum[:, T - 1:T]
    nb = jnp.floor((counts + (_TM - 1)) / _TM)
    il = lax.broadcasted_iota(jnp.int32, (_E, _E), 0)
    jl = lax.broadcasted_iota(jnp.int32, (_E, _E), 1)
    L8s = (jl < il).astype(jnp.float32)
    po = lax.dot_general(L8s, nb, (((1,), (0,)), ((), ())),
                         preferred_element_type=jnp.float32, precision=_PHI) * _TM
    psel = jnp.sum(po * oh, axis=0, keepdims=True)
    p = psel + rank_incl - 1.0
    p_ref[...] = p.astype(jnp.int32)
    mp_ref[...] = jnp.broadcast_to(lax.transpose(mp, (1, 0)),
                                   (T, mp_ref.shape[1]))

    maxb = be_ref.shape[1]
    nbi = po / _TM + nb
    jbf = lax.broadcasted_iota(jnp.int32, (_E, maxb), 1).astype(jnp.float32)
    becnt = jnp.sum((nbi <= jbf).astype(jnp.float32), axis=0, keepdims=True)
    be_ref[...] = jnp.minimum(becnt, float(_E - 1)).astype(jnp.int32)
    nblk_ref[...] = jnp.full((1, 1), 1.0, jnp.float32).astype(jnp.int32) * \
        jnp.sum(nb).astype(jnp.int32)

    pe = lax.dot_general(oh, mp, (((1,), (1,)), ((), ())),
                         preferred_element_type=jnp.float32, precision=_PHI)
    loss = jnp.sum(counts * pe) * (_LOSS_SCALE * _E / (T * float(T) * T))
    loss_ref[...] = jnp.full((1, 1), 1.0, jnp.float32) * loss


def _router_call(lgp, maxb):
    T = lgp.shape[1]
    return pl.pallas_call(
        _router_body,
        out_shape=(
            jax.ShapeDtypeStruct((1, T), jnp.int32),
            jax.ShapeDtypeStruct((T, 128), jnp.float32),
            jax.ShapeDtypeStruct((1, maxb), jnp.int32),
            jax.ShapeDtypeStruct((1, 1), jnp.int32),
            jax.ShapeDtypeStruct((1, 1), jnp.float32),
        ),
    )(lgp)


def _sc_wid():
    return lax.axis_index("s") * 2 + lax.axis_index("c")


def _dispatch_call(xt, p, mpb, padT):
    T, D = xt.shape
    NW = 32
    rpw = T // NW
    mesh = plsc.VectorSubcoreMesh(core_axis_name="c", subcore_axis_name="s")

    CH = 32
    nch = rpw // CH

    @functools.partial(
        pl.kernel, mesh=mesh,
        out_type=[jax.ShapeDtypeStruct((padT, D), jnp.float32),
                  jax.ShapeDtypeStruct((padT, 128), jnp.float32)],
        scratch_types=[pltpu.VMEM((rpw,), jnp.int32),
                       pltpu.VMEM((CH, D), jnp.float32),
                       pltpu.VMEM((CH, D), jnp.float32),
                       pltpu.VMEM((CH, 128), jnp.float32),
                       pltpu.VMEM((CH, 128), jnp.float32),
                       pltpu.SemaphoreType.DMA,
                       pltpu.SemaphoreType.DMA,
                       pltpu.SemaphoreType.DMA,
                       pltpu.SemaphoreType.DMA,
                       pltpu.SemaphoreType.DMA,
                       pltpu.SemaphoreType.DMA,
                       pltpu.SemaphoreType.DMA,
                       pltpu.SemaphoreType.DMA,
                       pltpu.SemaphoreType.DMA],
    )
    def k(x_hbm, p_hbm, mp_hbm, xs_hbm, mps_hbm, idx_v, rows0, rows1,
          mp0, mp1, li, lr0, lr1, lm0, lm1, sx0, sx1, sm0, sm1):
        b = _sc_wid() * rpw
        pltpu.async_copy(p_hbm.at[pl.ds(b, rpw)], idx_v, li).wait()
        rows = (rows0, rows1)
        mpv = (mp0, mp1)
        lrs = (lr0, lr1)
        lms = (lm0, lm1)
        sxs = (sx0, sx1)
        sms = (sm0, sm1)

        def loads(c):
            sl = c % 2
            return (
                pltpu.async_copy(x_hbm.at[pl.ds(b + c * CH, CH)],
                                 rows[sl], lrs[sl]),
                pltpu.async_copy(mp_hbm.at[pl.ds(b + c * CH, CH)],
                                 mpv[sl], lms[sl]))

        ld = [None] * nch
        sc = [None] * nch
        ld[0] = loads(0)
        ld[1] = loads(1)
        for c in range(nch):
            sl = c % 2
            ld[c][0].wait()
            ld[c][1].wait()
            isl = idx_v.at[pl.ds(c * CH, CH)]
            sc[c] = (pltpu.async_copy(rows[sl], xs_hbm.at[isl], sxs[sl]),
                     pltpu.async_copy(mpv[sl], mps_hbm.at[isl], sms[sl]))
            if c + 2 < nch:
                sc[c][0].wait()
                sc[c][1].wait()
                ld[c + 2] = loads(c + 2)
        for c in range(max(0, nch - 2), nch):
            sc[c][0].wait()
            sc[c][1].wait()

    return k(xt, p, mpb)


def _gmm_body(be_ref, nb_ref, xs_ref, w_ref, b_ref, mps_ref, ys_ref):
    @pl.when(pl.program_id(0) < nb_ref[0, 0])
    def _():
        acc = lax.dot_general(xs_ref[...], w_ref[0], (((1,), (1,)), ((), ())),
                              preferred_element_type=jnp.float32)
        ys_ref[...] = (acc + b_ref[0]) * mps_ref[:, :1]


def _gmm_call(be, nblk, xs, expert_W, expert_b, mps, maxb):
    padT, Dh = xs.shape
    D = expert_W.shape[1]
    grid_spec = pltpu.PrefetchScalarGridSpec(
        num_scalar_prefetch=2,
        grid=(maxb,),
        in_specs=[
            pl.BlockSpec((_TM, Dh),
                         lambda j, be, nb: (jnp.minimum(j, nb[0, 0] - 1), 0)),
            pl.BlockSpec((1, D, D), lambda j, be, nb: (be[0, j], 0, 0)),
            pl.BlockSpec((1, 1, D), lambda j, be, nb: (be[0, j], 0, 0)),
            pl.BlockSpec((_TM, 128),
                         lambda j, be, nb: (jnp.minimum(j, nb[0, 0] - 1), 0)),
        ],
        out_specs=pl.BlockSpec(
            (_TM, D), lambda j, be, nb: (jnp.minimum(j, nb[0, 0] - 1), 0)),
    )
    return pl.pallas_call(
        _gmm_body,
        grid_spec=grid_spec,
        out_shape=jax.ShapeDtypeStruct((padT, D), jnp.float32),
    )(be, nblk, xs, expert_W, expert_b, mps)


def _combine_call(ys, p):
    padT, D = ys.shape
    T = p.shape[0]
    NW = 32
    rpw = T // NW
    CH = 32
    mesh = plsc.VectorSubcoreMesh(core_axis_name="c", subcore_axis_name="s")

    @functools.partial(
        pl.kernel, mesh=mesh,
        out_type=jax.ShapeDtypeStruct((T, D), jnp.float32),
        scratch_types=[pltpu.VMEM((rpw,), jnp.int32),
                       pltpu.VMEM((CH, D), jnp.float32),
                       pltpu.VMEM((CH, D), jnp.float32),
                       pltpu.SemaphoreType.DMA,
                       pltpu.SemaphoreType.DMA,
                       pltpu.SemaphoreType.DMA,
                       pltpu.SemaphoreType.DMA,
                       pltpu.SemaphoreType.DMA],
    )
    def k(ys_hbm, p_hbm, out_hbm, idx_v, rows0, rows1, li, g0, g1, st0, st1):
        b = _sc_wid() * rpw
        pltpu.async_copy(p_hbm.at[pl.ds(b, rpw)], idx_v, li).wait()
        rows = (rows0, rows1)
        gsem = (g0, g1)
        ssem = (st0, st1)
        nch = rpw // CH
        gh = [None] * nch
        sh = [None] * nch
        gh[0] = pltpu.async_copy(ys_hbm.at[idx_v.at[pl.ds(0, CH)]], rows0, g0)
        for c in range(nch):
            sl = c % 2
            if c >= 1:
                sh[c - 1].wait()
            if c + 1 < nch:
                sl1 = (c + 1) % 2
                gh[c + 1] = pltpu.async_copy(
                    ys_hbm.at[idx_v.at[pl.ds((c + 1) * CH, CH)]],
                    rows[sl1], gsem[sl1])
            gh[c].wait()
            sh[c] = pltpu.async_copy(rows[sl], out_hbm.at[pl.ds(b + c * CH, CH)],
                                     ssem[sl])
        sh[nch - 1].wait()

    return k(ys, p)


def kernel(x, router_W, router_b, expert_W, expert_b):
    B, S, D = x.shape
    T = B * S
    maxb = T // _TM + _E - 1
    padT = maxb * _TM
    xt = x.reshape(T, D)
    logits = xt @ router_W.T + router_b

    p_row, mpb, be_row, nblk11, loss11 = _router_call(logits.T, maxb)
    p = p_row.reshape(T)

    xs, mps = _dispatch_call(xt, p, mpb, padT)
    ys = _gmm_call(be_row, nblk11, xs, expert_W, expert_b.reshape(_E, 1, D),
                   mps, maxb)
    out = _combine_call(ys, p)
    return out.reshape(B, S, D), loss11.reshape(())
